# speedup vs baseline: 5.3102x; 5.3102x over previous
import functools

import jax
import jax.numpy as jnp
from jax.experimental import pallas as pl
from jax.experimental.pallas import tpu as pltpu

KNN = 3
BATCH_OFF = 1000.0


def _knn_body(yoff_ref, ysq_ref, xsT_ref, xsq_ref, idx_ref):
    yc = yoff_ref[...]
    ysq = ysq_ref[...]
    xsT = xsT_ref[...]
    xsq = xsq_ref[...]
    dot = jax.lax.dot_general(yc, xsT, (((1,), (0,)), ((), ())),
                              preferred_element_type=jnp.float32)
    d2 = (ysq + xsq) - 2.0 * dot
    iota = jax.lax.broadcasted_iota(jnp.int32, d2.shape, 1)
    big = jnp.int32(2**30)
    inf = jnp.float32(jnp.inf)
    for k in range(KNN):
        m = jnp.min(d2, axis=1, keepdims=True)
        cand = jnp.where(d2 == m, iota, big)
        i = jnp.min(cand, axis=1)
        idx_ref[k, :] = i
        if k < KNN - 1:
            d2 = jnp.where(iota == i[:, None], inf, d2)


def _knn_topk_idx(off_y, ysq, xsT, xsq, block_rows=128):
    ny = off_y.shape[0]
    nx = xsT.shape[1]
    grid = (ny // block_rows,)
    return pl.pallas_call(
        _knn_body,
        grid=grid,
        in_specs=[
            pl.BlockSpec((block_rows, 3), lambda i: (i, 0)),
            pl.BlockSpec((block_rows, 1), lambda i: (i, 0)),
            pl.BlockSpec((3, nx), lambda i: (0, 0)),
            pl.BlockSpec((1, nx), lambda i: (0, 0)),
        ],
        out_specs=pl.BlockSpec((KNN, block_rows), lambda i: (0, i)),
        out_shape=jax.ShapeDtypeStruct((KNN, ny), jnp.int32),
    )(off_y, ysq, xsT, xsq)


def kernel(x, pos, batch, x_skip, pos_skip, batch_skip, W, b):
    off_x = pos + BATCH_OFF * batch[:, None].astype(pos.dtype)
    off_y = pos_skip + BATCH_OFF * batch_skip[:, None].astype(pos_skip.dtype)
    xsq = jnp.sum(off_x * off_x, axis=-1)
    ysq = jnp.sum(off_y * off_y, axis=-1)

    idx = _knn_topk_idx(off_y, ysq[:, None], off_x.T, xsq[None, :])
    idx_t = idx.T

    sel = off_x[idx_t]
    diff = off_y[:, None, :] - sel
    d2s = jnp.sum(diff * diff, axis=-1)
    w = 1.0 / jnp.clip(d2s, 1e-16)
    xf = x[idx_t]
    num = jnp.sum(w[..., None] * xf, axis=1)
    den = jnp.sum(w, axis=1, keepdims=True)
    xi = num / den
    h = jnp.concatenate([xi, x_skip], axis=1)
    out = h @ W + b
    return (out, pos_skip, batch_skip)


# speedup vs baseline: 8.1019x; 1.5257x over previous
import functools

import jax
import jax.numpy as jnp
from jax.experimental import pallas as pl
from jax.experimental.pallas import tpu as pltpu

KNN = 3
BATCH_OFF = 1000.0
_BIG = 2**30


def _insert3(state, bv, bi):
    m1, i1, m2, i2, m3, i3 = state
    t1 = bv < m1
    nm1 = jnp.where(t1, bv, m1)
    ni1 = jnp.where(t1, bi, i1)
    xv = jnp.where(t1, m1, bv)
    xi = jnp.where(t1, i1, bi)
    t2 = xv < m2
    nm2 = jnp.where(t2, xv, m2)
    ni2 = jnp.where(t2, xi, i2)
    yv = jnp.where(t2, m2, xv)
    yi = jnp.where(t2, i2, xi)
    t3 = yv < m3
    nm3 = jnp.where(t3, yv, m3)
    ni3 = jnp.where(t3, yi, i3)
    return (nm1, ni1, nm2, ni2, nm3, ni3)


def _knn_body(lo_ref, nt_ref, yc_ref, ysq_ref, xsT_ref, xsq_ref, idx_ref,
              *, block_rows, col_tile):
    blk = pl.program_id(0)
    lo = lo_ref[blk]
    ntiles = nt_ref[blk]
    yc = yc_ref[...]
    ysq = ysq_ref[...]

    r = block_rows
    inf = jnp.float32(jnp.inf)
    liota = jax.lax.broadcasted_iota(jnp.int32, (r, col_tile), 1)

    def tile_step(t, state):
        start = pl.multiple_of(lo + t * col_tile, 128)
        xsT = xsT_ref[:, pl.ds(start, col_tile)]
        xsq = xsq_ref[:, pl.ds(start, col_tile)]
        dot = jax.lax.dot_general(yc, xsT, (((1,), (0,)), ((), ())),
                                  preferred_element_type=jnp.float32)
        d2 = (ysq + xsq) - 2.0 * dot
        for _ in range(KNN):
            m = jnp.min(d2, axis=1, keepdims=True)
            cand = jnp.where(d2 == m, liota, jnp.int32(_BIG))
            il = jnp.min(cand, axis=1)
            state = _insert3(state, m[:, 0], il + start)
            d2 = jnp.where(liota == il[:, None], inf, d2)
        return state

    init = (jnp.full((r,), inf), jnp.full((r,), _BIG, jnp.int32),
            jnp.full((r,), inf), jnp.full((r,), _BIG, jnp.int32),
            jnp.full((r,), inf), jnp.full((r,), _BIG, jnp.int32))
    m1, i1, m2, i2, m3, i3 = jax.lax.fori_loop(0, ntiles, tile_step, init)
    idx_ref[0, :] = i1
    idx_ref[1, :] = i2
    idx_ref[2, :] = i3


def _knn_topk_idx(off_y, ysq, xsT_pad, xsq_pad, lo_arr, nt_arr,
                  block_rows=256, col_tile=512):
    ny = off_y.shape[0]
    nxp = xsT_pad.shape[1]
    grid = (ny // block_rows,)
    body = functools.partial(_knn_body, block_rows=block_rows,
                             col_tile=col_tile)
    return pl.pallas_call(
        body,
        grid_spec=pltpu.PrefetchScalarGridSpec(
            num_scalar_prefetch=2,
            grid=grid,
            in_specs=[
                pl.BlockSpec((block_rows, 3), lambda i, lo, nt: (i, 0)),
                pl.BlockSpec((block_rows, 1), lambda i, lo, nt: (i, 0)),
                pl.BlockSpec((3, nxp), lambda i, lo, nt: (0, 0)),
                pl.BlockSpec((1, nxp), lambda i, lo, nt: (0, 0)),
            ],
            out_specs=pl.BlockSpec((KNN, block_rows),
                                   lambda i, lo, nt: (0, i)),
        ),
        out_shape=jax.ShapeDtypeStruct((KNN, ny), jnp.int32),
    )(lo_arr, nt_arr, off_y, ysq, xsT_pad, xsq_pad)


def kernel(x, pos, batch, x_skip, pos_skip, batch_skip, W, b):
    nx = x.shape[0]
    ny = x_skip.shape[0]
    block_rows = 256
    col_tile = 512

    off_x = pos + BATCH_OFF * batch[:, None].astype(pos.dtype)
    off_y = pos_skip + BATCH_OFF * batch_skip[:, None].astype(pos_skip.dtype)
    xsq = jnp.sum(off_x * off_x, axis=-1)
    ysq = jnp.sum(off_y * off_y, axis=-1)

    nblk = ny // block_rows
    bs2 = batch_skip.reshape(nblk, block_rows)
    bmin = bs2[:, 0]
    bmax = bs2[:, -1]
    seg_lo = jnp.searchsorted(batch, bmin, side="left").astype(jnp.int32)
    seg_hi = jnp.searchsorted(batch, bmax, side="right").astype(jnp.int32)
    lo_arr = (seg_lo // 128) * 128
    nt_arr = (seg_hi - lo_arr + (col_tile - 1)) // col_tile

    xsT_pad = jnp.pad(off_x.T, ((0, 0), (0, col_tile)))
    xsq_pad = jnp.pad(xsq[None, :], ((0, 0), (0, col_tile)))

    idx = _knn_topk_idx(off_y, ysq[:, None], xsT_pad, xsq_pad,
                        lo_arr, nt_arr, block_rows, col_tile)
    idx_t = idx.T

    sel = off_x[idx_t]
    diff = off_y[:, None, :] - sel
    d2s = jnp.sum(diff * diff, axis=-1)
    w = 1.0 / jnp.clip(d2s, 1e-16)
    xf = x[idx_t]
    num = jnp.sum(w[..., None] * xf, axis=1)
    den = jnp.sum(w, axis=1, keepdims=True)
    xi = num / den
    h = jnp.concatenate([xi, x_skip], axis=1)
    out = h @ W + b
    return (out, pos_skip, batch_skip)


# speedup vs baseline: 13.4927x; 1.6654x over previous
import functools

import jax
import jax.numpy as jnp
from jax import lax
from jax.experimental import pallas as pl
from jax.experimental.pallas import tpu as pltpu
from jax.experimental.pallas import tpu_sc as plsc

KNN = 3
BATCH_OFF = 1000.0
_BIG = 2**30


def _insert3(state, cand):
    out = []
    e = cand
    for s in state:
        t = e[0] < s[0]
        keep = tuple(jnp.where(t, a, b) for a, b in zip(e, s))
        e = tuple(jnp.where(t, b, a) for a, b in zip(e, s))
        out.append(keep)
    return tuple(out)


def _knn_body(lo_ref, nt_ref, yc_ref, ysq_ref, xsT_ref, xsq_ref, idx_ref,
              *, block_rows, col_tile):
    blk = pl.program_id(0)
    lo = lo_ref[blk]
    ntiles = nt_ref[blk]
    yc = yc_ref[...]
    ysq = ysq_ref[...]

    r = block_rows
    inf = jnp.float32(jnp.inf)
    liota = jax.lax.broadcasted_iota(jnp.int32, (r, col_tile), 1)

    def tile_step(t, state):
        start = pl.multiple_of(lo + t * col_tile, 128)
        xsT = xsT_ref[:, pl.ds(start, col_tile)]
        xsq = xsq_ref[:, pl.ds(start, col_tile)]
        dot = jax.lax.dot_general(yc, xsT, (((1,), (0,)), ((), ())),
                                  preferred_element_type=jnp.float32)
        d2 = (ysq + xsq) - 2.0 * dot
        for k in range(KNN):
            m = jnp.min(d2, axis=1, keepdims=True)
            cand = jnp.where(d2 == m, liota, jnp.int32(_BIG))
            il = jnp.min(cand, axis=1)
            state = _insert3(state, (m[:, 0], il + start))
            if k < KNN - 1:
                d2 = jnp.where(liota == il[:, None], inf, d2)
        return state

    def entry():
        return (jnp.full((r,), inf), jnp.full((r,), _BIG, jnp.int32))

    state = jax.lax.fori_loop(0, ntiles, tile_step,
                              (entry(), entry(), entry()))
    for k in range(KNN):
        idx_ref[k, :] = state[k][1]


def _knn_topk(off_y, ysq, xsT_pad, xsq_pad, lo_arr, nt_arr,
              block_rows=256, col_tile=512):
    ny = off_y.shape[0]
    nxp = xsT_pad.shape[1]
    grid = (ny // block_rows,)
    body = functools.partial(_knn_body, block_rows=block_rows,
                             col_tile=col_tile)
    return pl.pallas_call(
        body,
        grid_spec=pltpu.PrefetchScalarGridSpec(
            num_scalar_prefetch=2,
            grid=grid,
            in_specs=[
                pl.BlockSpec((block_rows, 3), lambda i, lo, nt: (i, 0)),
                pl.BlockSpec((block_rows, 1), lambda i, lo, nt: (i, 0)),
                pl.BlockSpec((3, nxp), lambda i, lo, nt: (0, 0)),
                pl.BlockSpec((1, nxp), lambda i, lo, nt: (0, 0)),
            ],
            out_specs=pl.BlockSpec((KNN, block_rows),
                                   lambda i, lo, nt: (0, i)),
        ),
        out_shape=jax.ShapeDtypeStruct((KNN, ny), jnp.int32),
    )(lo_arr, nt_arr, off_y, ysq, xsT_pad, xsq_pad)


def _lane_bcast(v, lane):
    idx = jnp.full((16, 1), lane, jnp.int32)
    dn = lax.GatherDimensionNumbers(offset_dims=(), collapsed_slice_dims=(0,),
                                    start_index_map=(0,))
    return lax.gather(v, idx, dn, (1,),
                      mode=lax.GatherScatterMode.PROMISE_IN_BOUNDS)


def _interp_sc(x, oxp, oyp, idx_rows, chunk=128):
    nx, f = x.shape
    ny = idx_rows[0].shape[0]
    info = plsc.get_sparse_core_info()
    nw = info.num_cores * info.num_subcores
    rw = ny // nw
    nchunks = rw // chunk
    mesh = plsc.VectorSubcoreMesh(core_axis_name="c", subcore_axis_name="s")

    def body(x_h, oxp_h, oyp_h, i0_h, i1_h, i2_h, out_h,
             iv0, iv1, iv2, p0, p1, p2, yv, g0, g1, g2, ov, sem):
        wid = lax.axis_index("s") * info.num_cores + lax.axis_index("c")

        def chunk_body(ci, carry):
            base = wid * rw + ci * chunk
            pltpu.sync_copy(i0_h.at[pl.ds(base, chunk)], iv0)
            pltpu.sync_copy(i1_h.at[pl.ds(base, chunk)], iv1)
            pltpu.sync_copy(i2_h.at[pl.ds(base, chunk)], iv2)
            c0 = pltpu.async_copy(x_h.at[iv0], g0, sem)
            c1 = pltpu.async_copy(x_h.at[iv1], g1, sem)
            c2 = pltpu.async_copy(x_h.at[iv2], g2, sem)
            c3 = pltpu.async_copy(oxp_h.at[iv0], p0, sem)
            c4 = pltpu.async_copy(oxp_h.at[iv1], p1, sem)
            c5 = pltpu.async_copy(oxp_h.at[iv2], p2, sem)
            pltpu.sync_copy(oyp_h.at[pl.ds(base, chunk)], yv)
            for c in (c0, c1, c2, c3, c4, c5):
                c.wait()

            def row_body(r, c):
                yrow = yv[r, :]

                def wk(pref):
                    dd = yrow - pref[r, :]
                    s = dd * dd
                    d2s = (_lane_bcast(s, 0) + _lane_bcast(s, 1)) + _lane_bcast(s, 2)
                    return 1.0 / jnp.maximum(d2s, 1e-16)

                w0 = wk(p0)
                w1 = wk(p1)
                w2 = wk(p2)
                inv = 1.0 / ((w0 + w1) + w2)
                a0 = w0 * inv
                a1 = w1 * inv
                a2 = w2 * inv
                for fi in range(f // 16):
                    fs = pl.ds(fi * 16, 16)
                    ov[r, fs] = (a0 * g0[r, fs] + a1 * g1[r, fs]) + a2 * g2[r, fs]
                return c

            lax.fori_loop(0, chunk, row_body, 0)
            pltpu.sync_copy(ov, out_h.at[pl.ds(base, chunk)])
            return carry

        lax.fori_loop(0, nchunks, chunk_body, 0)

    return pl.kernel(
        body,
        out_type=jax.ShapeDtypeStruct((ny, f), jnp.float32),
        mesh=mesh,
        compiler_params=pltpu.CompilerParams(use_tc_tiling_on_sc=False),
        scratch_types=[
            pltpu.VMEM((chunk,), jnp.int32),
            pltpu.VMEM((chunk,), jnp.int32),
            pltpu.VMEM((chunk,), jnp.int32),
            pltpu.VMEM((chunk, 16), jnp.float32),
            pltpu.VMEM((chunk, 16), jnp.float32),
            pltpu.VMEM((chunk, 16), jnp.float32),
            pltpu.VMEM((chunk, 16), jnp.float32),
            pltpu.VMEM((chunk, f), jnp.float32),
            pltpu.VMEM((chunk, f), jnp.float32),
            pltpu.VMEM((chunk, f), jnp.float32),
            pltpu.VMEM((chunk, f), jnp.float32),
            pltpu.SemaphoreType.DMA,
        ],
    )(x, oxp, oyp, idx_rows[0], idx_rows[1], idx_rows[2])


def _lin_body(xi_ref, xs_ref, w1_ref, w2_ref, b_ref, o_ref):
    acc = jax.lax.dot_general(xi_ref[...], w1_ref[...],
                              (((1,), (0,)), ((), ())),
                              preferred_element_type=jnp.float32)
    acc = acc + jax.lax.dot_general(xs_ref[...], w2_ref[...],
                                    (((1,), (0,)), ((), ())),
                                    preferred_element_type=jnp.float32)
    o_ref[...] = acc + b_ref[...]


def _linear(xi, x_skip, W, b, block_rows=1024):
    ny, f = xi.shape
    w1 = W[:f]
    w2 = W[f:]
    b2d = b[None, :]
    grid = (ny // block_rows,)
    return pl.pallas_call(
        _lin_body,
        grid=grid,
        in_specs=[
            pl.BlockSpec((block_rows, f), lambda i: (i, 0)),
            pl.BlockSpec((block_rows, f), lambda i: (i, 0)),
            pl.BlockSpec((f, f), lambda i: (0, 0)),
            pl.BlockSpec((f, f), lambda i: (0, 0)),
            pl.BlockSpec((1, f), lambda i: (0, 0)),
        ],
        out_specs=pl.BlockSpec((block_rows, f), lambda i: (i, 0)),
        out_shape=jax.ShapeDtypeStruct((ny, f), jnp.float32),
    )(xi, x_skip, w1, w2, b2d)


def kernel(x, pos, batch, x_skip, pos_skip, batch_skip, W, b):
    nx = x.shape[0]
    ny = x_skip.shape[0]
    block_rows = 256
    col_tile = 512

    off_x = pos + BATCH_OFF * batch[:, None].astype(pos.dtype)
    off_y = pos_skip + BATCH_OFF * batch_skip[:, None].astype(pos_skip.dtype)
    xsq = jnp.sum(off_x * off_x, axis=-1)
    ysq = jnp.sum(off_y * off_y, axis=-1)

    nblk = ny // block_rows
    bs2 = batch_skip.reshape(nblk, block_rows)
    bmin = bs2[:, 0]
    bmax = bs2[:, -1]
    seg_lo = jnp.searchsorted(batch, bmin, side="left").astype(jnp.int32)
    seg_hi = jnp.searchsorted(batch, bmax, side="right").astype(jnp.int32)
    lo_arr = (seg_lo // 128) * 128
    nt_arr = (seg_hi - lo_arr + (col_tile - 1)) // col_tile

    xsT_pad = jnp.pad(off_x.T, ((0, 0), (0, col_tile)))
    xsq_pad = jnp.pad(xsq[None, :], ((0, 0), (0, col_tile)))

    idx = _knn_topk(off_y, ysq[:, None], xsT_pad, xsq_pad,
                    lo_arr, nt_arr, block_rows, col_tile)

    oxp = jnp.pad(off_x, ((0, 0), (0, 13)))
    oyp = jnp.pad(off_y, ((0, 0), (0, 13)))
    xi = _interp_sc(x, oxp, oyp, (idx[0], idx[1], idx[2]))

    out = _linear(xi, x_skip, W, b)
    return (out, pos_skip, batch_skip)


# speedup vs baseline: 19.2979x; 1.4302x over previous
import functools

import jax
import jax.numpy as jnp
from jax import lax
from jax.experimental import pallas as pl
from jax.experimental.pallas import tpu as pltpu
from jax.experimental.pallas import tpu_sc as plsc

KNN = 3
BATCH_OFF = 1000.0
_BIG = 2**30


def _insert3(state, cand):
    out = []
    e = cand
    for s in state:
        t = e[0] < s[0]
        keep = tuple(jnp.where(t, a, b) for a, b in zip(e, s))
        e = tuple(jnp.where(t, b, a) for a, b in zip(e, s))
        out.append(keep)
    return tuple(out)


def _knn_body(lo_ref, nt_ref, yc_ref, ysq_ref, xs_ref, xsq_ref, idx_ref,
              *, block_rows, col_tile):
    blk = pl.program_id(0)
    lo = lo_ref[blk]
    ntiles = nt_ref[blk]
    yc = yc_ref[...]
    ysq = ysq_ref[...]

    r = block_rows
    inf = jnp.float32(jnp.inf)
    siota = jax.lax.broadcasted_iota(jnp.int32, (col_tile, r), 0)

    def tile_step(t, state):
        start = pl.multiple_of(lo + t * col_tile, 128)
        xs = xs_ref[pl.ds(start, col_tile), :]
        xsq = xsq_ref[pl.ds(start, col_tile), :]
        dot = jax.lax.dot_general(xs, yc, (((1,), (1,)), ((), ())),
                                  preferred_element_type=jnp.float32)
        d2 = (ysq + xsq) - 2.0 * dot
        for k in range(KNN):
            m = jnp.min(d2, axis=0, keepdims=True)
            cand = jnp.where(d2 == m, siota, jnp.int32(_BIG))
            il = jnp.min(cand, axis=0, keepdims=True)
            state = _insert3(state, (m, il + start))
            if k < KNN - 1:
                d2 = jnp.where(siota == il, inf, d2)
        return state

    def entry():
        return (jnp.full((1, r), inf), jnp.full((1, r), _BIG, jnp.int32))

    state = jax.lax.fori_loop(0, ntiles, tile_step,
                              (entry(), entry(), entry()))
    for k in range(KNN):
        idx_ref[k, :] = state[k][1][0]


def _knn_topk(off_y, ysq, xs_pad, xsq_pad, lo_arr, nt_arr,
              block_rows=256, col_tile=512):
    ny = off_y.shape[0]
    nxp = xs_pad.shape[0]
    grid = (ny // block_rows,)
    body = functools.partial(_knn_body, block_rows=block_rows,
                             col_tile=col_tile)
    return pl.pallas_call(
        body,
        grid_spec=pltpu.PrefetchScalarGridSpec(
            num_scalar_prefetch=2,
            grid=grid,
            in_specs=[
                pl.BlockSpec((block_rows, 3), lambda i, lo, nt: (i, 0)),
                pl.BlockSpec((1, block_rows), lambda i, lo, nt: (0, i)),
                pl.BlockSpec((nxp, 3), lambda i, lo, nt: (0, 0)),
                pl.BlockSpec((nxp, 1), lambda i, lo, nt: (0, 0)),
            ],
            out_specs=pl.BlockSpec((KNN, block_rows),
                                   lambda i, lo, nt: (0, i)),
        ),
        out_shape=jax.ShapeDtypeStruct((KNN, ny), jnp.int32),
    )(lo_arr, nt_arr, off_y, ysq, xs_pad, xsq_pad)


def _lane_bcast(v, lane):
    idx = jnp.full((16, 1), lane, jnp.int32)
    dn = lax.GatherDimensionNumbers(offset_dims=(), collapsed_slice_dims=(0,),
                                    start_index_map=(0,))
    return lax.gather(v, idx, dn, (1,),
                      mode=lax.GatherScatterMode.PROMISE_IN_BOUNDS)


def _interp_sc(x, oxp, oyp, idx_rows, chunk=128):
    nx, f = x.shape
    ny = idx_rows[0].shape[0]
    info = plsc.get_sparse_core_info()
    nw = info.num_cores * info.num_subcores
    rw = ny // nw
    nchunks = rw // chunk
    mesh = plsc.VectorSubcoreMesh(core_axis_name="c", subcore_axis_name="s")

    def body(x_h, oxp_h, oyp_h, i0_h, i1_h, i2_h, out_h,
             iv0, iv1, iv2, p0, p1, p2, yv, g0, g1, g2, ov, sem):
        wid = lax.axis_index("s") * info.num_cores + lax.axis_index("c")

        def chunk_body(ci, carry):
            base = wid * rw + ci * chunk
            pltpu.sync_copy(i0_h.at[pl.ds(base, chunk)], iv0)
            pltpu.sync_copy(i1_h.at[pl.ds(base, chunk)], iv1)
            pltpu.sync_copy(i2_h.at[pl.ds(base, chunk)], iv2)
            c0 = pltpu.async_copy(x_h.at[iv0], g0, sem)
            c1 = pltpu.async_copy(x_h.at[iv1], g1, sem)
            c2 = pltpu.async_copy(x_h.at[iv2], g2, sem)
            c3 = pltpu.async_copy(oxp_h.at[iv0], p0, sem)
            c4 = pltpu.async_copy(oxp_h.at[iv1], p1, sem)
            c5 = pltpu.async_copy(oxp_h.at[iv2], p2, sem)
            pltpu.sync_copy(oyp_h.at[pl.ds(base, chunk)], yv)
            for c in (c0, c1, c2, c3, c4, c5):
                c.wait()

            def row_body(r, c):
                yrow = yv[r, :]

                def wk(pref):
                    dd = yrow - pref[r, :]
                    s = dd * dd
                    d2s = (_lane_bcast(s, 0) + _lane_bcast(s, 1)) + _lane_bcast(s, 2)
                    return 1.0 / jnp.maximum(d2s, 1e-16)

                w0 = wk(p0)
                w1 = wk(p1)
                w2 = wk(p2)
                inv = 1.0 / ((w0 + w1) + w2)
                a0 = w0 * inv
                a1 = w1 * inv
                a2 = w2 * inv
                for fi in range(f // 16):
                    fs = pl.ds(fi * 16, 16)
                    ov[r, fs] = (a0 * g0[r, fs] + a1 * g1[r, fs]) + a2 * g2[r, fs]
                return c

            lax.fori_loop(0, chunk, row_body, 0)
            pltpu.sync_copy(ov, out_h.at[pl.ds(base, chunk)])
            return carry

        lax.fori_loop(0, nchunks, chunk_body, 0)

    return pl.kernel(
        body,
        out_type=jax.ShapeDtypeStruct((ny, f), jnp.float32),
        mesh=mesh,
        compiler_params=pltpu.CompilerParams(use_tc_tiling_on_sc=False),
        scratch_types=[
            pltpu.VMEM((chunk,), jnp.int32),
            pltpu.VMEM((chunk,), jnp.int32),
            pltpu.VMEM((chunk,), jnp.int32),
            pltpu.VMEM((chunk, 16), jnp.float32),
            pltpu.VMEM((chunk, 16), jnp.float32),
            pltpu.VMEM((chunk, 16), jnp.float32),
            pltpu.VMEM((chunk, 16), jnp.float32),
            pltpu.VMEM((chunk, f), jnp.float32),
            pltpu.VMEM((chunk, f), jnp.float32),
            pltpu.VMEM((chunk, f), jnp.float32),
            pltpu.VMEM((chunk, f), jnp.float32),
            pltpu.SemaphoreType.DMA,
        ],
    )(x, oxp, oyp, idx_rows[0], idx_rows[1], idx_rows[2])


def _lin_body(xi_ref, xs_ref, w1_ref, w2_ref, b_ref, o_ref):
    acc = jax.lax.dot_general(xi_ref[...], w1_ref[...],
                              (((1,), (0,)), ((), ())),
                              preferred_element_type=jnp.float32)
    acc = acc + jax.lax.dot_general(xs_ref[...], w2_ref[...],
                                    (((1,), (0,)), ((), ())),
                                    preferred_element_type=jnp.float32)
    o_ref[...] = acc + b_ref[...]


def _linear(xi, x_skip, W, b, block_rows=1024):
    ny, f = xi.shape
    w1 = W[:f]
    w2 = W[f:]
    b2d = b[None, :]
    grid = (ny // block_rows,)
    return pl.pallas_call(
        _lin_body,
        grid=grid,
        in_specs=[
            pl.BlockSpec((block_rows, f), lambda i: (i, 0)),
            pl.BlockSpec((block_rows, f), lambda i: (i, 0)),
            pl.BlockSpec((f, f), lambda i: (0, 0)),
            pl.BlockSpec((f, f), lambda i: (0, 0)),
            pl.BlockSpec((1, f), lambda i: (0, 0)),
        ],
        out_specs=pl.BlockSpec((block_rows, f), lambda i: (i, 0)),
        out_shape=jax.ShapeDtypeStruct((ny, f), jnp.float32),
    )(xi, x_skip, w1, w2, b2d)


def kernel(x, pos, batch, x_skip, pos_skip, batch_skip, W, b):
    nx = x.shape[0]
    ny = x_skip.shape[0]
    block_rows = 256
    col_tile = 512

    off_x = pos + BATCH_OFF * batch[:, None].astype(pos.dtype)
    off_y = pos_skip + BATCH_OFF * batch_skip[:, None].astype(pos_skip.dtype)
    xsq = jnp.sum(off_x * off_x, axis=-1)
    ysq = jnp.sum(off_y * off_y, axis=-1)

    nblk = ny // block_rows
    bs2 = batch_skip.reshape(nblk, block_rows)
    bmin = bs2[:, 0]
    bmax = bs2[:, -1]
    seg_lo = jnp.searchsorted(batch, bmin, side="left").astype(jnp.int32)
    seg_hi = jnp.searchsorted(batch, bmax, side="right").astype(jnp.int32)
    lo_arr = (seg_lo // 128) * 128
    nt_arr = (seg_hi - lo_arr + (col_tile - 1)) // col_tile

    xs_pad = jnp.pad(off_x, ((0, col_tile), (0, 0)))
    xsq_pad = jnp.pad(xsq[:, None], ((0, col_tile), (0, 0)))

    idx = _knn_topk(off_y, ysq[None, :], xs_pad, xsq_pad,
                    lo_arr, nt_arr, block_rows, col_tile)

    oxp = jnp.pad(off_x, ((0, 0), (0, 13)))
    oyp = jnp.pad(off_y, ((0, 0), (0, 13)))
    xi = _interp_sc(x, oxp, oyp, (idx[0], idx[1], idx[2]))

    out = _linear(xi, x_skip, W, b)
    return (out, pos_skip, batch_skip)
